# Initial kernel scaffold; baseline (speedup 1.0000x reference)
import jax
import jax.numpy as jnp
from jax.experimental import pallas as pl


def kernel(input, edge_list, W, a):
    raise NotImplementedError("write your pallas kernel here")


# speedup vs baseline: 13.7994x; 13.7994x over previous
import functools

import jax
import jax.numpy as jnp
from jax import lax
from jax.experimental import pallas as pl
from jax.experimental.pallas import tpu as pltpu
from jax.experimental.pallas import tpu_sc as plsc

N = 10000
E = 320000
D = 128

NC = 2
NS = 16
NW = NC * NS
C = 128
E_PAD = ((E + NW * C - 1) // (NW * C)) * (NW * C)
PT = E_PAD // NW
NCH = PT // C
N_PAD = 10112
RPT = N_PAD // NS


def _mm_body(x_ref, w_ref, a1_ref, a2_ref, h_ref, s1_ref, s2_ref):
    h = jnp.dot(x_ref[...], w_ref[...], preferred_element_type=jnp.float32)
    h_ref[...] = h
    s1_ref[...] = jnp.dot(h, a1_ref[...], preferred_element_type=jnp.float32)
    s2_ref[...] = jnp.dot(h, a2_ref[...], preferred_element_type=jnp.float32)


def _mm(x, W, a1, a2):
    B = 2000
    grid = (N // B,)
    return pl.pallas_call(
        _mm_body,
        grid=grid,
        in_specs=[
            pl.BlockSpec((B, D), lambda i: (i, 0)),
            pl.BlockSpec((D, D), lambda i: (0, 0)),
            pl.BlockSpec((D, 1), lambda i: (0, 0)),
            pl.BlockSpec((D, 1), lambda i: (0, 0)),
        ],
        out_specs=[
            pl.BlockSpec((B, D), lambda i: (i, 0)),
            pl.BlockSpec((B, 1), lambda i: (i, 0)),
            pl.BlockSpec((B, 1), lambda i: (i, 0)),
        ],
        out_shape=[
            jax.ShapeDtypeStruct((N, D), jnp.float32),
            jax.ShapeDtypeStruct((N, 1), jnp.float32),
            jax.ShapeDtypeStruct((N, 1), jnp.float32),
        ],
    )(x, W, a1, a2)


def _edge_body(h_hbm, s1_hbm, s2_hbm, src_hbm, dst_hbm,
               acc_out, den_out,
               s1t, s2t, src_v, dst_v, w_v, rows_v, acc_sh, den_sh, gsem):
    cid = lax.axis_index("c")
    sid = lax.axis_index("s")
    wid = cid * NS + sid

    def _zrow(i, carry):
        for f in range(D // 16):
            rows_v[i, pl.ds(f * 16, 16)] = jnp.zeros((16,), jnp.float32)
        return carry
    lax.fori_loop(0, C, _zrow, 0)

    r0 = sid * RPT
    for t in range(RPT // C):
        pltpu.sync_copy(rows_v, acc_sh.at[pl.ds(r0 + t * C, C)])
    rem = RPT % C
    if rem:
        pltpu.sync_copy(rows_v.at[pl.ds(0, rem)],
                        acc_sh.at[pl.ds(r0 + (RPT // C) * C, rem)])

    @pl.when(sid == 0)
    def _():
        def _zden(i, carry):
            s1t[pl.ds(i * 16, 16)] = jnp.zeros((16,), jnp.float32)
            return carry
        lax.fori_loop(0, N_PAD // 16, _zden, 0)
        pltpu.sync_copy(s1t, den_sh)

    pltpu.sync_copy(s1_hbm, s1t)
    pltpu.sync_copy(s2_hbm, s2t)
    plsc.subcore_barrier()

    def _chunk(k, carry):
        base = wid * PT + k * C
        pltpu.sync_copy(src_hbm.at[pl.ds(base, C)], src_v)
        pltpu.sync_copy(dst_hbm.at[pl.ds(base, C)], dst_v)
        gcp = pltpu.async_copy(h_hbm.at[dst_v], rows_v, gsem)
        for j in range(C // 16):
            s16 = src_v[pl.ds(j * 16, 16)]
            d16 = dst_v[pl.ds(j * 16, 16)]
            v = plsc.load_gather(s1t, [s16]) + plsc.load_gather(s2t, [d16])
            e = jnp.where(v >= 0, v, 0.2 * v)
            w_v[pl.ds(j * 16, 16)] = jnp.exp(e)
        gcp.wait()

        def _scale(g, carry2):
            w16 = w_v[pl.ds(g * 16, 16)]
            for i in range(16):
                ws = w16[i]
                r = g * 16 + i
                for f in range(D // 16):
                    rows_v[r, pl.ds(f * 16, 16)] = (
                        rows_v[r, pl.ds(f * 16, 16)] * ws)
            return carry2
        lax.fori_loop(0, C // 16, _scale, 0)

        pltpu.sync_copy(rows_v, acc_sh.at[src_v], add=True)
        pltpu.sync_copy(w_v, den_sh.at[src_v], add=True)
        return carry
    lax.fori_loop(0, NCH, _chunk, 0)

    plsc.subcore_barrier()

    pltpu.sync_copy(acc_sh.at[pl.ds(r0, RPT)], acc_out.at[cid, pl.ds(r0, RPT)])
    @pl.when(sid == 0)
    def _():
        pltpu.sync_copy(den_sh, den_out.at[cid])


@functools.partial(jax.jit, static_argnames=())
def _edge_sc(h, s1p, s2p, srcp, dstp):
    mesh = plsc.VectorSubcoreMesh(core_axis_name="c", subcore_axis_name="s")
    f = pl.kernel(
        _edge_body,
        out_type=[
            jax.ShapeDtypeStruct((NC, N_PAD, D), jnp.float32),
            jax.ShapeDtypeStruct((NC, N_PAD), jnp.float32),
        ],
        mesh=mesh,
        scratch_types=[
            pltpu.VMEM((N_PAD,), jnp.float32),
            pltpu.VMEM((N_PAD,), jnp.float32),
            pltpu.VMEM((C,), jnp.int32),
            pltpu.VMEM((C,), jnp.int32),
            pltpu.VMEM((C,), jnp.float32),
            pltpu.VMEM((C, D), jnp.float32),
            pltpu.VMEM_SHARED((N_PAD, D), jnp.float32),
            pltpu.VMEM_SHARED((N_PAD,), jnp.float32),
            pltpu.SemaphoreType.DMA,
        ],
        compiler_params=pltpu.CompilerParams(needs_layout_passes=False),
    )
    return f(h, s1p, s2p, srcp, dstp)


def _fin_body(a0_ref, a1_ref, d0_ref, d1_ref, o_ref):
    num = a0_ref[...] + a1_ref[...]
    den = d0_ref[...] + d1_ref[...]
    safe = jnp.where(den == 0.0, 1.0, den)
    r = num / safe
    out = jnp.where(r > 0.0, r, jnp.exp(jnp.minimum(r, 0.0)) - 1.0)
    o_ref[...] = jnp.where(den == 0.0, 0.0, out)


def _finish(a0, a1, d0, d1):
    B = 2000
    return pl.pallas_call(
        _fin_body,
        grid=(N // B,),
        in_specs=[
            pl.BlockSpec((B, D), lambda i: (i, 0)),
            pl.BlockSpec((B, D), lambda i: (i, 0)),
            pl.BlockSpec((B, 1), lambda i: (i, 0)),
            pl.BlockSpec((B, 1), lambda i: (i, 0)),
        ],
        out_specs=pl.BlockSpec((B, D), lambda i: (i, 0)),
        out_shape=jax.ShapeDtypeStruct((N, D), jnp.float32),
    )(a0, a1, d0, d1)


def kernel(input, edge_list, W, a):
    h, s1, s2 = _mm(input, W, a[:D], a[D:])
    zpad = jnp.zeros((N_PAD - N,), jnp.float32)
    s1p = jnp.concatenate([s1[:, 0], zpad])
    s2p = jnp.concatenate([s2[:, 0], zpad])
    srcp = jnp.concatenate(
        [edge_list[0], jnp.full((E_PAD - E,), N, jnp.int32)])
    dstp = jnp.concatenate(
        [edge_list[1], jnp.zeros((E_PAD - E,), jnp.int32)])
    acc, den = _edge_sc(h, s1p, s2p, srcp, dstp)
    return _finish(acc[0, :N], acc[1, :N],
                   den[0, :N, None], den[1, :N, None])
